# baseline (speedup 1.0000x reference)
import functools

import jax
import jax.numpy as jnp
from jax import lax
from jax.experimental import pallas as pl
from jax.experimental.pallas import tpu as pltpu
from jax.experimental.pallas import tpu_sc as plsc

T = 4096
D = 4096
E = 64
H = 1024
TOP_K = 8
N_CAND = 16

BT = 512

_PREC = jax.lax.Precision.DEFAULT

NW = 32
TPW = T // NW
L = 16


def _dense_block(x_ref, wr_ref, br_ref, w1_ref, b1_ref, w2_ref, b2_ref,
                 probs_ref, bvh_ref, wcat_ref, w2bf_ref):
    @pl.when(pl.program_id(0) == 0)
    def _convert_weights():
        wcat_ref[:, :H] = w1_ref[...].astype(jnp.bfloat16)
        wcat_ref[:, H:] = wr_ref[...].astype(jnp.bfloat16)
        w2bf_ref[...] = w2_ref[...].astype(jnp.bfloat16)

    x = x_ref[...].astype(jnp.bfloat16)
    y = jnp.dot(x, wcat_ref[...], preferred_element_type=jnp.float32,
                precision=_PREC)

    logits = y[:, H:] + br_ref[...]
    lt = logits.T
    m = jnp.max(lt, axis=0, keepdims=True)
    e = jnp.exp(lt - m)
    pt = e / jnp.sum(e, axis=0, keepdims=True)
    probs_ref[...] = pt.T

    h1 = jnp.maximum(y[:, :H] + b1_ref[...], 0.0).astype(jnp.bfloat16)
    bvh_ref[...] = jnp.dot(h1, w2bf_ref[...],
                           preferred_element_type=jnp.float32,
                           precision=_PREC) + b2_ref[...]


def _dense(x, W_router, b_router, W_bvh1, b_bvh1, W_bvh2, b_bvh2):
    tt = x.shape[0]
    full = lambda i: (0, 0)
    return pl.pallas_call(
        _dense_block,
        grid=(tt // BT,),
        in_specs=[
            pl.BlockSpec((BT, D), lambda i: (i, 0)),
            pl.BlockSpec((D, E), full),
            pl.BlockSpec((1, E), full),
            pl.BlockSpec((D, H), full),
            pl.BlockSpec((1, H), full),
            pl.BlockSpec((H, E), full),
            pl.BlockSpec((1, E), full),
        ],
        out_specs=[
            pl.BlockSpec((BT, E), lambda i: (i, 0)),
            pl.BlockSpec((BT, E), lambda i: (i, 0)),
        ],
        out_shape=[
            jax.ShapeDtypeStruct((tt, E), jnp.float32),
            jax.ShapeDtypeStruct((tt, E), jnp.float32),
        ],
        scratch_shapes=[
            pltpu.VMEM((D, H + E), jnp.bfloat16),
            pltpu.VMEM((H, E), jnp.bfloat16),
        ],
    )(x, W_router, b_router.reshape(1, E), W_bvh1, b_bvh1.reshape(1, H),
      W_bvh2, b_bvh2.reshape(1, E))


def _sort_desc(k, v):
    return plsc.sort_key_val(k, v, descending=True)


def _merge_desc(ka, va, kb, vb):
    rkb = lax.rev(kb, (0,))
    rvb = lax.rev(vb, (0,))
    take = ka >= rkb
    return jnp.where(take, ka, rkb), jnp.where(take, va, rvb)


def _sc_tail_body(tpw, probs_hbm, bvh_hbm, pes_hbm, w_hbm, i_hbm,
                  pv, bv, wv, iv, pesv):
    wid = lax.axis_index("s") * 2 + lax.axis_index("c")
    base = wid * tpw
    pltpu.sync_copy(probs_hbm.at[pl.ds(base, tpw)], pv)
    pltpu.sync_copy(bvh_hbm.at[pl.ds(base, tpw)], bv)
    pltpu.sync_copy(pes_hbm, pesv)

    lane = lax.iota(jnp.int32, L)

    def token(t, carry):
        b = [bv[t, pl.ds(L * i, L)] for i in range(E // L)]
        iot = [lane + L * i for i in range(E // L)]
        sb = [_sort_desc(b[i], iot[i])[0] for i in range(4)]
        m1, _ = _merge_desc(sb[0], lane, sb[1], lane)
        m2, _ = _merge_desc(sb[2], lane, sb[3], lane)
        m1 = _sort_desc(m1, lane)[0]
        m2 = _sort_desc(m2, lane)[0]
        t16, _ = _merge_desc(m1, lane, m2, lane)
        thr = jnp.min(t16)
        pm = [jnp.where(b[i] >= thr, pv[t, pl.ds(L * i, L)], -1.0)
              for i in range(4)]
        kv = [_sort_desc(pm[i], iot[i]) for i in range(4)]
        k1, v1 = _merge_desc(*kv[0], *kv[1])
        k2, v2 = _merge_desc(*kv[2], *kv[3])
        k1, v1 = _sort_desc(k1, v1)
        k2, v2 = _sort_desc(k2, v2)
        k3, v3 = _merge_desc(k1, v1, k2, v2)
        k3, v3 = _sort_desc(k3, v3)
        sel = lane < TOP_K
        s = jnp.sum(jnp.where(sel, k3, 0.0))
        scale = plsc.load_gather(pesv, [v3])
        wv[t, :] = jnp.where(sel, (k3 / s) * scale, 0.0)
        iv[t, :] = jnp.where(sel, v3, 0)
        return carry

    lax.fori_loop(0, tpw, token, 0)

    pltpu.sync_copy(wv, w_hbm.at[pl.ds(base, tpw)])
    pltpu.sync_copy(iv, i_hbm.at[pl.ds(base, tpw)])


def _sc_tail(probs, bvh, pes):
    tt = probs.shape[0]
    tpw = tt // NW
    mesh = plsc.VectorSubcoreMesh(core_axis_name="c", subcore_axis_name="s")
    f = pl.kernel(
        functools.partial(_sc_tail_body, tpw),
        mesh=mesh,
        compiler_params=pltpu.CompilerParams(needs_layout_passes=False),
        out_type=[
            jax.ShapeDtypeStruct((tt, L), jnp.float32),
            jax.ShapeDtypeStruct((tt, L), jnp.int32),
        ],
        scratch_types=[
            pltpu.VMEM((tpw, E), jnp.float32),
            pltpu.VMEM((tpw, E), jnp.float32),
            pltpu.VMEM((tpw, L), jnp.float32),
            pltpu.VMEM((tpw, L), jnp.int32),
            pltpu.VMEM((E,), jnp.float32),
        ],
    )
    return f(probs, bvh, pes)


@jax.jit
def kernel(hidden_states, W_router, b_router, W_bvh1, b_bvh1, W_bvh2, b_bvh2,
           per_expert_scale):
    halves = []
    for h in range(2):
        xh = hidden_states.reshape(T, D)[h * (T // 2):(h + 1) * (T // 2)]
        probs, bvh = _dense(xh, W_router, b_router, W_bvh1, b_bvh1, W_bvh2,
                            b_bvh2)
        w16, i16 = _sc_tail(probs, bvh, per_expert_scale)
        halves.append((probs, w16, i16))
    probs = jnp.concatenate([halves[0][0], halves[1][0]], axis=0)
    w16 = jnp.concatenate([halves[0][1], halves[1][1]], axis=0)
    i16 = jnp.concatenate([halves[0][2], halves[1][2]], axis=0)
    return (probs, w16[:, :TOP_K], i16[:, :TOP_K])


# speedup vs baseline: 1.5866x; 1.5866x over previous
import functools

import jax
import jax.numpy as jnp
from jax import lax
from jax.experimental import pallas as pl
from jax.experimental.pallas import tpu as pltpu
from jax.experimental.pallas import tpu_sc as plsc

T = 4096
D = 4096
E = 64
H = 1024
TOP_K = 8
N_CAND = 16

BT = 512

_PREC = jax.lax.Precision.DEFAULT

NW = 32
TPW = T // NW
L = 16


def _dense_block(x_ref, wr_ref, br_ref, w1_ref, b1_ref, w2_ref, b2_ref,
                 probs_ref, bvh_ref, wcat_ref, w2bf_ref):
    @pl.when(pl.program_id(0) == 0)
    def _convert_weights():
        wcat_ref[:, :H] = w1_ref[...].astype(jnp.bfloat16)
        wcat_ref[:, H:] = wr_ref[...].astype(jnp.bfloat16)
        w2bf_ref[...] = w2_ref[...].astype(jnp.bfloat16)

    x = x_ref[...].astype(jnp.bfloat16)
    y = jnp.dot(x, wcat_ref[...], preferred_element_type=jnp.float32,
                precision=_PREC)

    logits = y[:, H:] + br_ref[...]
    lt = logits.T
    m = jnp.max(lt, axis=0, keepdims=True)
    e = jnp.exp(lt - m)
    pt = e / jnp.sum(e, axis=0, keepdims=True)
    probs_ref[...] = pt.T

    h1 = jnp.maximum(y[:, :H] + b1_ref[...], 0.0).astype(jnp.bfloat16)
    bvh_ref[...] = jnp.dot(h1, w2bf_ref[...],
                           preferred_element_type=jnp.float32,
                           precision=_PREC) + b2_ref[...]


def _dense(x, W_router, b_router, W_bvh1, b_bvh1, W_bvh2, b_bvh2):
    tt = x.shape[0]
    full = lambda i: (0, 0)
    return pl.pallas_call(
        _dense_block,
        grid=(tt // BT,),
        in_specs=[
            pl.BlockSpec((BT, D), lambda i: (i, 0)),
            pl.BlockSpec((D, E), full),
            pl.BlockSpec((1, E), full),
            pl.BlockSpec((D, H), full),
            pl.BlockSpec((1, H), full),
            pl.BlockSpec((H, E), full),
            pl.BlockSpec((1, E), full),
        ],
        out_specs=[
            pl.BlockSpec((BT, E), lambda i: (i, 0)),
            pl.BlockSpec((BT, E), lambda i: (i, 0)),
        ],
        out_shape=[
            jax.ShapeDtypeStruct((tt, E), jnp.float32),
            jax.ShapeDtypeStruct((tt, E), jnp.float32),
        ],
        scratch_shapes=[
            pltpu.VMEM((D, H + E), jnp.bfloat16),
            pltpu.VMEM((H, E), jnp.bfloat16),
        ],
    )(x, W_router, b_router.reshape(1, E), W_bvh1, b_bvh1.reshape(1, H),
      W_bvh2, b_bvh2.reshape(1, E))


def _sort_desc(k, v):
    return plsc.sort_key_val(k, v, descending=True)


def _merge_desc(ka, va, kb, vb):
    rkb = lax.rev(kb, (0,))
    rvb = lax.rev(vb, (0,))
    take = ka >= rkb
    return jnp.where(take, ka, rkb), jnp.where(take, va, rvb)


def _sc_tail_body(tpw, probs_hbm, bvh_hbm, pes_hbm, w_hbm, i_hbm,
                  pv, bv, wv, iv, pesv):
    wid = lax.axis_index("s") * 2 + lax.axis_index("c")
    base = wid * tpw
    pltpu.sync_copy(probs_hbm.at[pl.ds(base, tpw)], pv)
    pltpu.sync_copy(bvh_hbm.at[pl.ds(base, tpw)], bv)
    pltpu.sync_copy(pes_hbm, pesv)

    lane = lax.iota(jnp.int32, L)

    def _one_token(t):
        b = [bv[t, pl.ds(L * i, L)] for i in range(E // L)]
        iot = [lane + L * i for i in range(E // L)]
        sb = [_sort_desc(b[i], iot[i])[0] for i in range(4)]
        m1, _ = _merge_desc(sb[0], lane, sb[1], lane)
        m2, _ = _merge_desc(sb[2], lane, sb[3], lane)
        m1 = _sort_desc(m1, lane)[0]
        m2 = _sort_desc(m2, lane)[0]
        t16, _ = _merge_desc(m1, lane, m2, lane)
        thr = jnp.min(t16)
        pm = [jnp.where(b[i] >= thr, pv[t, pl.ds(L * i, L)], -1.0)
              for i in range(4)]
        kv = [_sort_desc(pm[i], iot[i]) for i in range(4)]
        k1, v1 = _merge_desc(*kv[0], *kv[1])
        k2, v2 = _merge_desc(*kv[2], *kv[3])
        k1, v1 = _sort_desc(k1, v1)
        k2, v2 = _sort_desc(k2, v2)
        k3, v3 = _merge_desc(k1, v1, k2, v2)
        k3, v3 = _sort_desc(k3, v3)
        sel = lane < TOP_K
        s = jnp.sum(jnp.where(sel, k3, 0.0))
        scale = plsc.load_gather(pesv, [v3])
        wv[t, :] = jnp.where(sel, (k3 / s) * scale, 0.0)
        iv[t, :] = jnp.where(sel, v3, 0)

    def token(t2, carry):
        _one_token(t2 * 2)
        _one_token(t2 * 2 + 1)
        return carry

    lax.fori_loop(0, tpw // 2, token, 0)

    pltpu.sync_copy(wv, w_hbm.at[pl.ds(base, tpw)])
    pltpu.sync_copy(iv, i_hbm.at[pl.ds(base, tpw)])


def _sc_tail(probs, bvh, pes):
    tt = probs.shape[0]
    tpw = tt // NW
    mesh = plsc.VectorSubcoreMesh(core_axis_name="c", subcore_axis_name="s")
    f = pl.kernel(
        functools.partial(_sc_tail_body, tpw),
        mesh=mesh,
        compiler_params=pltpu.CompilerParams(needs_layout_passes=False),
        out_type=[
            jax.ShapeDtypeStruct((tt, L), jnp.float32),
            jax.ShapeDtypeStruct((tt, L), jnp.int32),
        ],
        scratch_types=[
            pltpu.VMEM((tpw, E), jnp.float32),
            pltpu.VMEM((tpw, E), jnp.float32),
            pltpu.VMEM((tpw, L), jnp.float32),
            pltpu.VMEM((tpw, L), jnp.int32),
            pltpu.VMEM((E,), jnp.float32),
        ],
    )
    return f(probs, bvh, pes)


@jax.jit
def kernel(hidden_states, W_router, b_router, W_bvh1, b_bvh1, W_bvh2, b_bvh2,
           per_expert_scale):
    x = hidden_states.reshape(T, D)
    probs, bvh = _dense(x, W_router, b_router, W_bvh1, b_bvh1, W_bvh2,
                        b_bvh2)
    w16, i16 = _sc_tail(probs, bvh, per_expert_scale)
    return (probs, w16[:, :TOP_K], i16[:, :TOP_K])
